# baseline (speedup 1.0000x reference)
import functools

import numpy as np
import jax
import jax.numpy as jnp
from jax import lax
from jax.experimental import pallas as pl
from jax.experimental.pallas import tpu as pltpu
from jax.experimental.pallas import tpu_sc as plsc

_FIELD_DIMS = [100000] * 26
_F = 26
_D = 16
_B = 16384
_TOTAL = int(sum(_FIELD_DIMS))
_OFFSETS = np.concatenate(([0], np.cumsum(_FIELD_DIMS)[:-1])).astype(np.int32)

_M = np.zeros((_F * _D, _D), np.float32)
for _f in range(_F):
    for _d in range(_D):
        _M[_f * _D + _d, _d] = 1.0

_NW = 32
_CH = 128
_PER_W = (_B * _F) // _NW
_NCH = _PER_W // _CH

_PAD_IDX = ((np.arange(_NW)[:, None, None] * 409
             + np.arange(2)[None, :, None] * 211
             + np.arange(_CH)[None, None, :] * 97) % 100000).astype(np.int32)

_NBLK4 = 4
_PAD_DST = ((np.arange(_NW)[:, None, None] * 256
             + np.arange(2)[None, :, None] * 128
             + np.arange(_CH)[None, None, :]) * 8 + 7).astype(np.int32)

_LMASK = np.zeros((_NBLK4, 1, 128), np.float32)
for _r in range(_NBLK4):
    for _l in range(8):
        if 4 * _l + _r < _F:
            _LMASK[_r, 0, _l * _D:(_l + 1) * _D] = 1.0

_KMAP = np.full((_NBLK4, 128), _F * _D, np.int32)
for _r in range(_NBLK4):
    for _l in range(8):
        _f2 = 4 * _l + _r
        if _f2 < _F:
            for _d in range(_D):
                _KMAP[_r, _l * _D + _d] = _f2 * _D + _d

_NN = np.arange(_B * _F, dtype=np.int64)
_DSTG = ((_NN % _F % 4) * (_B * 8) + (_NN // _F) * 8
         + (_NN % _F) // 4).astype(np.int32)
_DST3 = np.concatenate([_DSTG.reshape(_NW, _NCH, _CH), _PAD_DST], axis=1)


_TC = 32768
_TG = (_TOTAL + _TC - 1) // _TC
_C8 = _TC // 8
_TROWS = _TG * _C8


def _transpose_table(tab_t):
    def body(x_ref, i_ref, y_ref):
        x = x_ref[...]
        xcat = jnp.concatenate([x[:, u * _C8:(u + 1) * _C8] for u in range(8)],
                               axis=0)
        base = pl.program_id(0) * _TC
        u8 = lax.broadcasted_iota(jnp.int32, (128, _C8), 0) // _D
        col = lax.broadcasted_iota(jnp.int32, (128, _C8), 1)
        xcat = jnp.where(base + u8 * _C8 + col < _TOTAL, xcat, 0.0)
        y_ref[...] = lax.dot_general(xcat, i_ref[...],
                                     (((0,), (0,)), ((), ())),
                                     preferred_element_type=jnp.float32)

    return pl.pallas_call(
        body,
        grid=(_TG,),
        in_specs=[
            pl.BlockSpec((_D, _TC), lambda j: (0, j)),
            pl.BlockSpec((128, 128), lambda j: (0, 0)),
        ],
        out_specs=pl.BlockSpec((_C8, 128), lambda j: (j, 0)),
        out_shape=jax.ShapeDtypeStruct((_TROWS, 128), jnp.float32),
    )(tab_t, jnp.eye(128, dtype=jnp.float32))


def _sc_gather(idx3, dst3, embed_table):
    mesh = plsc.VectorSubcoreMesh(core_axis_name="c", subcore_axis_name="s")

    @functools.partial(
        pl.kernel,
        out_type=jax.ShapeDtypeStruct((_NBLK4 * _B * 8, _D), jnp.float32),
        mesh=mesh,
        compiler_params=pltpu.CompilerParams(use_tc_tiling_on_sc=False),
        scratch_types=[
            pltpu.VMEM((_NCH + 2, _CH), jnp.int32),
            pltpu.VMEM((_NCH + 2, _CH), jnp.int32),
            pltpu.VMEM((2, _CH, _D), jnp.float32),
            pltpu.SemaphoreType.DMA,
            pltpu.SemaphoreType.DMA,
            pltpu.SemaphoreType.DMA,
        ],
    )
    def k(idx_hbm, dst_hbm, tab_hbm, oute_hbm, idx_v, dst_v, ebuf,
          se0, se1, so):
        w = lax.axis_index("s") * 2 + lax.axis_index("c")
        pltpu.sync_copy(idx_hbm.at[w], idx_v)
        pltpu.sync_copy(dst_hbm.at[w], dst_v)
        se = (se0, se1)

        def g_start(j, s):
            pltpu.async_copy(tab_hbm.at[idx_v.at[j]], ebuf.at[s], se[s])

        def g_wait(j, s):
            pltpu.make_async_copy(tab_hbm.at[idx_v.at[j]], ebuf.at[s], se[s]).wait()

        g_start(0, 0)
        g_start(1, 1)

        def body(j2, carry):
            for s in (0, 1):
                j = j2 * 2 + s
                g_wait(j, s)
                pltpu.async_copy(ebuf.at[s], oute_hbm.at[dst_v.at[j]], so).wait()
                g_start(j + 2, s)
            return carry

        lax.fori_loop(0, _NCH // 2, body, 0)
        g_wait(_NCH, 0)
        g_wait(_NCH + 1, 1)

    return k(idx3, dst3, embed_table)


_BLK = 512
_NBLK = _B // _BLK
_H1 = 128
_H2 = 64
_IN = _F * _D
_INC = _IN + _D


def _pass_a(o4, linval, w14, b1, lbias):
    def body(o_ref, lv_ref, w_ref, b_ref, lb_ref, a1_ref, fl_ref, st_ref):
        i = pl.program_id(0)
        lane = lax.broadcasted_iota(jnp.int32, (_BLK, 128), 1)
        acc = jnp.zeros((_BLK, _H1 + _D), jnp.float32)
        e2 = jnp.zeros((_BLK, 1), jnp.float32)
        for r in range(_NBLK4):
            o = o_ref[r, :, :]
            o = jnp.where(4 * (lane // _D) + r < _F, o, 0.0)
            acc += jnp.dot(o, w_ref[r, :, :],
                           preferred_element_type=jnp.float32)
            e2 += jnp.sum(o * o, axis=1, keepdims=True)
        a1 = acc[:, :_H1] + b_ref[...]
        a1_ref[...] = a1
        s = acc[:, _H1:]
        fm = 0.5 * (jnp.sum(s * s, axis=1, keepdims=True) - e2)
        lin = jnp.sum(lv_ref[...], axis=1, keepdims=True) + lb_ref[0, 0]
        fl_ref[...] = fm + lin

        @pl.when(i == 0)
        def _():
            st_ref[...] = jnp.zeros_like(st_ref)

        st_ref[0:1, :] += jnp.sum(a1, axis=0, keepdims=True)
        st_ref[1:2, :] += jnp.sum(a1 * a1, axis=0, keepdims=True)

    return pl.pallas_call(
        body,
        grid=(_NBLK,),
        in_specs=[
            pl.BlockSpec((_NBLK4, _BLK, 128), lambda i: (0, i, 0)),
            pl.BlockSpec((_BLK, _F), lambda i: (i, 0)),
            pl.BlockSpec((_NBLK4, 128, _H1 + _D), lambda i: (0, 0, 0)),
            pl.BlockSpec((1, _H1), lambda i: (0, 0)),
            pl.BlockSpec((1, 1), lambda i: (0, 0), memory_space=pltpu.SMEM),
        ],
        out_specs=[
            pl.BlockSpec((_BLK, _H1), lambda i: (i, 0)),
            pl.BlockSpec((_BLK, 1), lambda i: (i, 0)),
            pl.BlockSpec((2, _H1), lambda i: (0, 0)),
        ],
        out_shape=[
            jax.ShapeDtypeStruct((_B, _H1), jnp.float32),
            jax.ShapeDtypeStruct((_B, 1), jnp.float32),
            jax.ShapeDtypeStruct((2, _H1), jnp.float32),
        ],
    )(o4, linval, w14, b1, lbias)


def _pass_b(a1, st1, g1, bt1, w2, b2):
    def body(a_ref, st_ref, g_ref, bt_ref, w_ref, b_ref, a2_ref, st2_ref):
        i = pl.program_id(0)
        m = st_ref[0:1, :] * (1.0 / _B)
        v = st_ref[1:2, :] * (1.0 / _B) - m * m
        rstd = lax.rsqrt(v + 1e-5)
        scale = g_ref[...] * rstd
        shift = bt_ref[...] - m * scale
        h = jnp.maximum(a_ref[...] * scale + shift, 0.0)
        a2 = jnp.dot(h, w_ref[...], preferred_element_type=jnp.float32) + b_ref[...]
        a2_ref[...] = a2

        @pl.when(i == 0)
        def _():
            st2_ref[...] = jnp.zeros_like(st2_ref)

        st2_ref[0:1, :] += jnp.sum(a2, axis=0, keepdims=True)
        st2_ref[1:2, :] += jnp.sum(a2 * a2, axis=0, keepdims=True)

    return pl.pallas_call(
        body,
        grid=(_NBLK,),
        in_specs=[
            pl.BlockSpec((_BLK, _H1), lambda i: (i, 0)),
            pl.BlockSpec((2, _H1), lambda i: (0, 0)),
            pl.BlockSpec((1, _H1), lambda i: (0, 0)),
            pl.BlockSpec((1, _H1), lambda i: (0, 0)),
            pl.BlockSpec((_H1, _H2), lambda i: (0, 0)),
            pl.BlockSpec((1, _H2), lambda i: (0, 0)),
        ],
        out_specs=[
            pl.BlockSpec((_BLK, _H2), lambda i: (i, 0)),
            pl.BlockSpec((2, _H2), lambda i: (0, 0)),
        ],
        out_shape=[
            jax.ShapeDtypeStruct((_B, _H2), jnp.float32),
            jax.ShapeDtypeStruct((2, _H2), jnp.float32),
        ],
    )(a1, st1, g1, bt1, w2, b2)


def _pass_c(a2, st2, g2, bt2, w3row, fl, c0):
    def body(a_ref, st_ref, g_ref, bt_ref, w_ref, fl_ref, c0_ref, y_ref):
        m = st_ref[0:1, :] * (1.0 / _B)
        v = st_ref[1:2, :] * (1.0 / _B) - m * m
        rstd = lax.rsqrt(v + 1e-5)
        scale = g_ref[...] * rstd
        shift = bt_ref[...] - m * scale
        h = jnp.maximum(a_ref[...] * scale + shift, 0.0)
        mlp = jnp.sum(h * w_ref[...], axis=1, keepdims=True)
        y = mlp + fl_ref[...] + c0_ref[0, 0]
        y_ref[...] = 1.0 / (1.0 + jnp.exp(-y))

    return pl.pallas_call(
        body,
        grid=(_NBLK,),
        in_specs=[
            pl.BlockSpec((_BLK, _H2), lambda i: (i, 0)),
            pl.BlockSpec((2, _H2), lambda i: (0, 0)),
            pl.BlockSpec((1, _H2), lambda i: (0, 0)),
            pl.BlockSpec((1, _H2), lambda i: (0, 0)),
            pl.BlockSpec((1, _H2), lambda i: (0, 0)),
            pl.BlockSpec((_BLK, 1), lambda i: (i, 0)),
            pl.BlockSpec((1, 1), lambda i: (0, 0), memory_space=pltpu.SMEM),
        ],
        out_specs=pl.BlockSpec((_BLK, 1), lambda i: (i, 0)),
        out_shape=jax.ShapeDtypeStruct((_B, 1), jnp.float32),
    )(a2, st2, g2, bt2, w3row, fl, c0)


def kernel(x, embed_table, linear_table, linear_bias,
           W1, b1, g1, bt1, W2, b2, g2, bt2, W3, b3):
    idx = (x.astype(jnp.int32) + jnp.asarray(_OFFSETS)[None, :])
    idxq = ((idx // _TC) * _TC + (idx % _C8) * 8 + (idx % _TC) // _C8)
    idx3 = idxq.reshape(_NW, _NCH, _CH)
    idx3 = jnp.concatenate([idx3, jnp.asarray(_PAD_IDX)], axis=1)

    table_rm = _transpose_table(embed_table.T).reshape(_TROWS * 8, _D)
    o_flat = _sc_gather(idx3, jnp.asarray(_DST3), table_rm)
    o4 = o_flat.reshape(_NBLK4, _B, 128)
    lin1d = jnp.reshape(linear_table.T, (-1,))
    linval = jnp.take(lin1d, idx.reshape(-1), axis=0).reshape(_B, _F)

    w1cat = jnp.concatenate(
        [jnp.concatenate([W1, jnp.asarray(_M)], axis=1),
         jnp.zeros((1, _H1 + _D), jnp.float32)], axis=0)
    w14 = jnp.take(w1cat, jnp.asarray(_KMAP), axis=0)
    a1, fl, st1 = _pass_a(o4, linval, w14, b1.reshape(1, _H1),
                          linear_bias.reshape(1, 1))
    a2, st2 = _pass_b(a1, st1, g1.reshape(1, _H1), bt1.reshape(1, _H1),
                      W2, b2.reshape(1, _H2))
    c0 = b3.reshape(1, 1)
    y = _pass_c(a2, st2, g2.reshape(1, _H2), bt2.reshape(1, _H2),
                W3.reshape(1, _H2), fl, c0)
    return y.reshape(_B)


# speedup vs baseline: 1.0526x; 1.0526x over previous
import functools

import numpy as np
import jax
import jax.numpy as jnp
from jax import lax
from jax.experimental import pallas as pl
from jax.experimental.pallas import tpu as pltpu
from jax.experimental.pallas import tpu_sc as plsc

_FIELD_DIMS = [100000] * 26
_F = 26
_D = 16
_B = 16384
_TOTAL = int(sum(_FIELD_DIMS))
_OFFSETS = np.concatenate(([0], np.cumsum(_FIELD_DIMS)[:-1])).astype(np.int32)

_M = np.zeros((_F * _D, _D), np.float32)
for _f in range(_F):
    for _d in range(_D):
        _M[_f * _D + _d, _d] = 1.0

_NW = 32
_CH = 128
_PER_W = (_B * _F) // _NW
_NCH = _PER_W // _CH

_PAD_IDX = ((np.arange(_NW)[:, None, None] * 409
             + np.arange(2)[None, :, None] * 211
             + np.arange(_CH)[None, None, :] * 97) % 100000).astype(np.int32)

_NBLK4 = 4
_PAD_DST = ((np.arange(_NW)[:, None, None] * 256
             + np.arange(2)[None, :, None] * 128
             + np.arange(_CH)[None, None, :]) * 8 + 7).astype(np.int32)

_LMASK = np.zeros((_NBLK4, 1, 128), np.float32)
for _r in range(_NBLK4):
    for _l in range(8):
        if 4 * _l + _r < _F:
            _LMASK[_r, 0, _l * _D:(_l + 1) * _D] = 1.0

_KMAP = np.full((_NBLK4, 128), _F * _D, np.int32)
for _r in range(_NBLK4):
    for _l in range(8):
        _f2 = 4 * _l + _r
        if _f2 < _F:
            for _d in range(_D):
                _KMAP[_r, _l * _D + _d] = _f2 * _D + _d

_NN = np.arange(_B * _F, dtype=np.int64)
_DSTG = ((_NN % _F % 4) * (_B * 8) + (_NN // _F) * 8
         + (_NN % _F) // 4).astype(np.int32)
_DST3 = np.concatenate([_DSTG.reshape(_NW, _NCH, _CH), _PAD_DST], axis=1)


_TC = 32768
_TG = (_TOTAL + _TC - 1) // _TC
_C8 = _TC // 8
_TROWS = _TG * _C8


def _transpose_table(tab_t):
    def body(x_ref, i_ref, y_ref):
        x = x_ref[...]
        xcat = jnp.concatenate([x[:, u * _C8:(u + 1) * _C8] for u in range(8)],
                               axis=0)
        base = pl.program_id(0) * _TC
        u8 = lax.broadcasted_iota(jnp.int32, (128, _C8), 0) // _D
        col = lax.broadcasted_iota(jnp.int32, (128, _C8), 1)
        xcat = jnp.where(base + u8 * _C8 + col < _TOTAL, xcat, 0.0)
        y_ref[...] = lax.dot_general(xcat, i_ref[...],
                                     (((0,), (0,)), ((), ())),
                                     preferred_element_type=jnp.float32)

    return pl.pallas_call(
        body,
        grid=(_TG,),
        in_specs=[
            pl.BlockSpec((_D, _TC), lambda j: (0, j)),
            pl.BlockSpec((128, 128), lambda j: (0, 0)),
        ],
        out_specs=pl.BlockSpec((_C8, 128), lambda j: (j, 0)),
        out_shape=jax.ShapeDtypeStruct((_TROWS, 128), jnp.float32),
    )(tab_t, jnp.eye(128, dtype=jnp.float32))


def _sc_gather(idx3, dst3, embed_table):
    mesh = plsc.VectorSubcoreMesh(core_axis_name="c", subcore_axis_name="s")

    @functools.partial(
        pl.kernel,
        out_type=jax.ShapeDtypeStruct((_NBLK4 * _B * 8, _D), jnp.float32),
        mesh=mesh,
        compiler_params=pltpu.CompilerParams(use_tc_tiling_on_sc=False),
        scratch_types=[
            pltpu.VMEM((_NCH + 2, _CH), jnp.int32),
            pltpu.VMEM((_NCH + 2, _CH), jnp.int32),
            pltpu.VMEM((2, _CH, _D), jnp.float32),
            pltpu.SemaphoreType.DMA,
            pltpu.SemaphoreType.DMA,
            pltpu.SemaphoreType.DMA,
        ],
    )
    def k(idx_hbm, dst_hbm, tab_hbm, oute_hbm, idx_v, dst_v, ebuf,
          se0, se1, so):
        w = lax.axis_index("s") * 2 + lax.axis_index("c")
        pltpu.sync_copy(idx_hbm.at[w], idx_v)
        pltpu.sync_copy(dst_hbm.at[w], dst_v)
        se = (se0, se1)

        def g_start(j, s):
            pltpu.async_copy(tab_hbm.at[idx_v.at[j]], ebuf.at[s], se[s])

        def g_wait(j, s):
            pltpu.make_async_copy(tab_hbm.at[idx_v.at[j]], ebuf.at[s], se[s]).wait()

        g_start(0, 0)
        g_start(1, 1)

        def body(j2, carry):
            for s in (0, 1):
                j = j2 * 2 + s
                g_wait(j, s)
                pltpu.async_copy(ebuf.at[s], oute_hbm.at[dst_v.at[j]], so).wait()
                g_start(j + 2, s)
            return carry

        lax.fori_loop(0, _NCH // 2, body, 0)
        g_wait(_NCH, 0)
        g_wait(_NCH + 1, 1)

    return k(idx3, dst3, embed_table)


_BLK = 1024
_NBLK = _B // _BLK
_H1 = 128
_H2 = 64
_IN = _F * _D
_INC = _IN + _D


def _pass_a(o4, linval, w14, b1, lbias):
    def body(o_ref, lv_ref, w_ref, b_ref, lb_ref, a1_ref, fl_ref, st_ref):
        i = pl.program_id(0)
        lane = lax.broadcasted_iota(jnp.int32, (_BLK, 128), 1)
        acc = jnp.zeros((_BLK, _H1 + _D), jnp.float32)
        e2 = jnp.zeros((_BLK, 1), jnp.float32)
        for r in range(_NBLK4):
            o = o_ref[r, :, :]
            o = jnp.where(4 * (lane // _D) + r < _F, o, 0.0)
            acc += jnp.dot(o, w_ref[r, :, :],
                           preferred_element_type=jnp.float32)
            e2 += jnp.sum(o * o, axis=1, keepdims=True)
        a1 = acc[:, :_H1] + b_ref[...]
        a1_ref[...] = a1
        s = acc[:, _H1:]
        fm = 0.5 * (jnp.sum(s * s, axis=1, keepdims=True) - e2)
        lin = jnp.sum(lv_ref[...], axis=1, keepdims=True) + lb_ref[0, 0]
        fl_ref[...] = fm + lin

        @pl.when(i == 0)
        def _():
            st_ref[...] = jnp.zeros_like(st_ref)

        st_ref[0:1, :] += jnp.sum(a1, axis=0, keepdims=True)
        st_ref[1:2, :] += jnp.sum(a1 * a1, axis=0, keepdims=True)

    return pl.pallas_call(
        body,
        grid=(_NBLK,),
        in_specs=[
            pl.BlockSpec((_NBLK4, _BLK, 128), lambda i: (0, i, 0)),
            pl.BlockSpec((_BLK, _F), lambda i: (i, 0)),
            pl.BlockSpec((_NBLK4, 128, _H1 + _D), lambda i: (0, 0, 0)),
            pl.BlockSpec((1, _H1), lambda i: (0, 0)),
            pl.BlockSpec((1, 1), lambda i: (0, 0), memory_space=pltpu.SMEM),
        ],
        out_specs=[
            pl.BlockSpec((_BLK, _H1), lambda i: (i, 0)),
            pl.BlockSpec((_BLK, 1), lambda i: (i, 0)),
            pl.BlockSpec((2, _H1), lambda i: (0, 0)),
        ],
        out_shape=[
            jax.ShapeDtypeStruct((_B, _H1), jnp.float32),
            jax.ShapeDtypeStruct((_B, 1), jnp.float32),
            jax.ShapeDtypeStruct((2, _H1), jnp.float32),
        ],
    )(o4, linval, w14, b1, lbias)


def _pass_b(a1, st1, g1, bt1, w2, b2):
    def body(a_ref, st_ref, g_ref, bt_ref, w_ref, b_ref, a2_ref, st2_ref):
        i = pl.program_id(0)
        m = st_ref[0:1, :] * (1.0 / _B)
        v = st_ref[1:2, :] * (1.0 / _B) - m * m
        rstd = lax.rsqrt(v + 1e-5)
        scale = g_ref[...] * rstd
        shift = bt_ref[...] - m * scale
        h = jnp.maximum(a_ref[...] * scale + shift, 0.0)
        a2 = jnp.dot(h, w_ref[...], preferred_element_type=jnp.float32) + b_ref[...]
        a2_ref[...] = a2

        @pl.when(i == 0)
        def _():
            st2_ref[...] = jnp.zeros_like(st2_ref)

        st2_ref[0:1, :] += jnp.sum(a2, axis=0, keepdims=True)
        st2_ref[1:2, :] += jnp.sum(a2 * a2, axis=0, keepdims=True)

    return pl.pallas_call(
        body,
        grid=(_NBLK,),
        in_specs=[
            pl.BlockSpec((_BLK, _H1), lambda i: (i, 0)),
            pl.BlockSpec((2, _H1), lambda i: (0, 0)),
            pl.BlockSpec((1, _H1), lambda i: (0, 0)),
            pl.BlockSpec((1, _H1), lambda i: (0, 0)),
            pl.BlockSpec((_H1, _H2), lambda i: (0, 0)),
            pl.BlockSpec((1, _H2), lambda i: (0, 0)),
        ],
        out_specs=[
            pl.BlockSpec((_BLK, _H2), lambda i: (i, 0)),
            pl.BlockSpec((2, _H2), lambda i: (0, 0)),
        ],
        out_shape=[
            jax.ShapeDtypeStruct((_B, _H2), jnp.float32),
            jax.ShapeDtypeStruct((2, _H2), jnp.float32),
        ],
    )(a1, st1, g1, bt1, w2, b2)


def _pass_c(a2, st2, g2, bt2, w3row, fl, c0):
    def body(a_ref, st_ref, g_ref, bt_ref, w_ref, fl_ref, c0_ref, y_ref):
        m = st_ref[0:1, :] * (1.0 / _B)
        v = st_ref[1:2, :] * (1.0 / _B) - m * m
        rstd = lax.rsqrt(v + 1e-5)
        scale = g_ref[...] * rstd
        shift = bt_ref[...] - m * scale
        h = jnp.maximum(a_ref[...] * scale + shift, 0.0)
        mlp = jnp.sum(h * w_ref[...], axis=1, keepdims=True)
        y = mlp + fl_ref[...] + c0_ref[0, 0]
        y_ref[...] = 1.0 / (1.0 + jnp.exp(-y))

    return pl.pallas_call(
        body,
        grid=(_NBLK,),
        in_specs=[
            pl.BlockSpec((_BLK, _H2), lambda i: (i, 0)),
            pl.BlockSpec((2, _H2), lambda i: (0, 0)),
            pl.BlockSpec((1, _H2), lambda i: (0, 0)),
            pl.BlockSpec((1, _H2), lambda i: (0, 0)),
            pl.BlockSpec((1, _H2), lambda i: (0, 0)),
            pl.BlockSpec((_BLK, 1), lambda i: (i, 0)),
            pl.BlockSpec((1, 1), lambda i: (0, 0), memory_space=pltpu.SMEM),
        ],
        out_specs=pl.BlockSpec((_BLK, 1), lambda i: (i, 0)),
        out_shape=jax.ShapeDtypeStruct((_B, 1), jnp.float32),
    )(a2, st2, g2, bt2, w3row, fl, c0)


def kernel(x, embed_table, linear_table, linear_bias,
           W1, b1, g1, bt1, W2, b2, g2, bt2, W3, b3):
    idx = (x.astype(jnp.int32) + jnp.asarray(_OFFSETS)[None, :])
    idxq = ((idx // _TC) * _TC + (idx % _C8) * 8 + (idx % _TC) // _C8)
    idx3 = idxq.reshape(_NW, _NCH, _CH)
    idx3 = jnp.concatenate([idx3, jnp.asarray(_PAD_IDX)], axis=1)

    table_rm = _transpose_table(embed_table.T).reshape(_TROWS * 8, _D)
    o_flat = _sc_gather(idx3, jnp.asarray(_DST3), table_rm)
    o4 = o_flat.reshape(_NBLK4, _B, 128)
    linval = jnp.take(linear_table, idx.reshape(-1), axis=0).reshape(_B, _F)

    w1cat = jnp.concatenate(
        [jnp.concatenate([W1, jnp.asarray(_M)], axis=1),
         jnp.zeros((1, _H1 + _D), jnp.float32)], axis=0)
    w14 = jnp.take(w1cat, jnp.asarray(_KMAP), axis=0)
    a1, fl, st1 = _pass_a(o4, linval, w14, b1.reshape(1, _H1),
                          linear_bias.reshape(1, 1))
    a2, st2 = _pass_b(a1, st1, g1.reshape(1, _H1), bt1.reshape(1, _H1),
                      W2, b2.reshape(1, _H2))
    c0 = b3.reshape(1, 1)
    y = _pass_c(a2, st2, g2.reshape(1, _H2), bt2.reshape(1, _H2),
                W3.reshape(1, _H2), fl, c0)
    return y.reshape(_B)


# speedup vs baseline: 1.1326x; 1.0761x over previous
import functools

import numpy as np
import jax
import jax.numpy as jnp
from jax import lax
from jax.experimental import pallas as pl
from jax.experimental.pallas import tpu as pltpu
from jax.experimental.pallas import tpu_sc as plsc

_FIELD_DIMS = [100000] * 26
_F = 26
_D = 16
_B = 16384
_TOTAL = int(sum(_FIELD_DIMS))
_OFFSETS = np.concatenate(([0], np.cumsum(_FIELD_DIMS)[:-1])).astype(np.int32)

_M = np.zeros((_F * _D, _D), np.float32)
for _f in range(_F):
    for _d in range(_D):
        _M[_f * _D + _d, _d] = 1.0

_NW = 32
_CH = 128
_PER_W = (_B * _F) // _NW
_NCH = _PER_W // _CH

_PAD_IDX = ((np.arange(_NW)[:, None, None] * 409
             + np.arange(2)[None, :, None] * 211
             + np.arange(_CH)[None, None, :] * 97) % 100000).astype(np.int32)

_NBLK4 = 4
_PAD_DST = ((np.arange(_NW)[:, None, None] * 256
             + np.arange(2)[None, :, None] * 128
             + np.arange(_CH)[None, None, :]) * 8 + 7).astype(np.int32)

_LMASK = np.zeros((_NBLK4, 1, 128), np.float32)
for _r in range(_NBLK4):
    for _l in range(8):
        if 4 * _l + _r < _F:
            _LMASK[_r, 0, _l * _D:(_l + 1) * _D] = 1.0

_KMAP = np.full((_NBLK4, 128), _F * _D, np.int32)
for _r in range(_NBLK4):
    for _l in range(8):
        _f2 = 4 * _l + _r
        if _f2 < _F:
            for _d in range(_D):
                _KMAP[_r, _l * _D + _d] = _f2 * _D + _d

_NN = np.arange(_B * _F, dtype=np.int64)
_DSTG = ((_NN % _F % 4) * (_B * 8) + (_NN // _F) * 8
         + (_NN % _F) // 4).astype(np.int32)
_DST3 = np.concatenate([_DSTG.reshape(_NW, _NCH, _CH), _PAD_DST], axis=1)


_TC = 65536
_TG = (_TOTAL + _TC - 1) // _TC
_C8 = _TC // 8
_TROWS = _TG * _C8


def _transpose_table(tab_t):
    def body(x_ref, i_ref, y_ref):
        x = x_ref[...]
        xcat = jnp.concatenate([x[:, u * _C8:(u + 1) * _C8] for u in range(8)],
                               axis=0)
        base = pl.program_id(0) * _TC
        u8 = lax.broadcasted_iota(jnp.int32, (128, _C8), 0) // _D
        col = lax.broadcasted_iota(jnp.int32, (128, _C8), 1)
        xcat = jnp.where(base + u8 * _C8 + col < _TOTAL, xcat, 0.0)
        y_ref[...] = lax.dot_general(xcat, i_ref[...],
                                     (((0,), (0,)), ((), ())),
                                     preferred_element_type=jnp.float32)

    return pl.pallas_call(
        body,
        grid=(_TG,),
        in_specs=[
            pl.BlockSpec((_D, _TC), lambda j: (0, j)),
            pl.BlockSpec((128, 128), lambda j: (0, 0)),
        ],
        out_specs=pl.BlockSpec((_C8, 128), lambda j: (j, 0)),
        out_shape=jax.ShapeDtypeStruct((_TROWS, 128), jnp.float32),
    )(tab_t, jnp.eye(128, dtype=jnp.float32))


def _sc_gather(idx3, dst3, embed_table):
    mesh = plsc.VectorSubcoreMesh(core_axis_name="c", subcore_axis_name="s")

    @functools.partial(
        pl.kernel,
        out_type=jax.ShapeDtypeStruct((_NBLK4 * _B * 8, _D), jnp.float32),
        mesh=mesh,
        compiler_params=pltpu.CompilerParams(use_tc_tiling_on_sc=False),
        scratch_types=[
            pltpu.VMEM((_NCH + 2, _CH), jnp.int32),
            pltpu.VMEM((_NCH + 2, _CH), jnp.int32),
            pltpu.VMEM((2, _CH, _D), jnp.float32),
            pltpu.SemaphoreType.DMA,
            pltpu.SemaphoreType.DMA,
            pltpu.SemaphoreType.DMA,
        ],
    )
    def k(idx_hbm, dst_hbm, tab_hbm, oute_hbm, idx_v, dst_v, ebuf,
          se0, se1, so):
        w = lax.axis_index("s") * 2 + lax.axis_index("c")
        pltpu.sync_copy(idx_hbm.at[w], idx_v)
        pltpu.sync_copy(dst_hbm.at[w], dst_v)
        se = (se0, se1)

        def g_start(j, s):
            pltpu.async_copy(tab_hbm.at[idx_v.at[j]], ebuf.at[s], se[s])

        def g_wait(j, s):
            pltpu.make_async_copy(tab_hbm.at[idx_v.at[j]], ebuf.at[s], se[s]).wait()

        g_start(0, 0)
        g_start(1, 1)

        def body(j2, carry):
            for s in (0, 1):
                j = j2 * 2 + s
                g_wait(j, s)
                pltpu.async_copy(ebuf.at[s], oute_hbm.at[dst_v.at[j]], so).wait()
                g_start(j + 2, s)
            return carry

        lax.fori_loop(0, _NCH // 2, body, 0)
        g_wait(_NCH, 0)
        g_wait(_NCH + 1, 1)

    return k(idx3, dst3, embed_table)


_BLK = 2048
_NBLK = _B // _BLK
_H1 = 128
_H2 = 64
_IN = _F * _D
_INC = _IN + _D


def _pass_a(o4, linval, w14, b1, lbias):
    def body(o_ref, lv_ref, w_ref, b_ref, lb_ref, a1_ref, fl_ref, st_ref):
        i = pl.program_id(0)
        lane = lax.broadcasted_iota(jnp.int32, (_BLK, 128), 1)
        acc = jnp.zeros((_BLK, _H1 + _D), jnp.float32)
        e2 = jnp.zeros((_BLK, 1), jnp.float32)
        for r in range(_NBLK4):
            o = o_ref[r, :, :]
            o = jnp.where(4 * (lane // _D) + r < _F, o, 0.0)
            acc += jnp.dot(o, w_ref[r, :, :],
                           preferred_element_type=jnp.float32)
            e2 += jnp.sum(o * o, axis=1, keepdims=True)
        a1 = acc[:, :_H1] + b_ref[...]
        a1_ref[...] = a1
        s = acc[:, _H1:]
        fm = 0.5 * (jnp.sum(s * s, axis=1, keepdims=True) - e2)
        lin = jnp.sum(lv_ref[...], axis=1, keepdims=True) + lb_ref[0, 0]
        fl_ref[...] = fm + lin

        @pl.when(i == 0)
        def _():
            st_ref[...] = jnp.zeros_like(st_ref)

        st_ref[0:1, :] += jnp.sum(a1, axis=0, keepdims=True)
        st_ref[1:2, :] += jnp.sum(a1 * a1, axis=0, keepdims=True)

    return pl.pallas_call(
        body,
        grid=(_NBLK,),
        in_specs=[
            pl.BlockSpec((_NBLK4, _BLK, 128), lambda i: (0, i, 0)),
            pl.BlockSpec((_BLK, _F), lambda i: (i, 0)),
            pl.BlockSpec((_NBLK4, 128, _H1 + _D), lambda i: (0, 0, 0)),
            pl.BlockSpec((1, _H1), lambda i: (0, 0)),
            pl.BlockSpec((1, 1), lambda i: (0, 0), memory_space=pltpu.SMEM),
        ],
        out_specs=[
            pl.BlockSpec((_BLK, _H1), lambda i: (i, 0)),
            pl.BlockSpec((_BLK, 1), lambda i: (i, 0)),
            pl.BlockSpec((2, _H1), lambda i: (0, 0)),
        ],
        out_shape=[
            jax.ShapeDtypeStruct((_B, _H1), jnp.float32),
            jax.ShapeDtypeStruct((_B, 1), jnp.float32),
            jax.ShapeDtypeStruct((2, _H1), jnp.float32),
        ],
    )(o4, linval, w14, b1, lbias)


def _pass_b(a1, st1, g1, bt1, w2, b2):
    def body(a_ref, st_ref, g_ref, bt_ref, w_ref, b_ref, a2_ref, st2_ref):
        i = pl.program_id(0)
        m = st_ref[0:1, :] * (1.0 / _B)
        v = st_ref[1:2, :] * (1.0 / _B) - m * m
        rstd = lax.rsqrt(v + 1e-5)
        scale = g_ref[...] * rstd
        shift = bt_ref[...] - m * scale
        h = jnp.maximum(a_ref[...] * scale + shift, 0.0)
        a2 = jnp.dot(h, w_ref[...], preferred_element_type=jnp.float32) + b_ref[...]
        a2_ref[...] = a2

        @pl.when(i == 0)
        def _():
            st2_ref[...] = jnp.zeros_like(st2_ref)

        st2_ref[0:1, :] += jnp.sum(a2, axis=0, keepdims=True)
        st2_ref[1:2, :] += jnp.sum(a2 * a2, axis=0, keepdims=True)

    return pl.pallas_call(
        body,
        grid=(_NBLK,),
        in_specs=[
            pl.BlockSpec((_BLK, _H1), lambda i: (i, 0)),
            pl.BlockSpec((2, _H1), lambda i: (0, 0)),
            pl.BlockSpec((1, _H1), lambda i: (0, 0)),
            pl.BlockSpec((1, _H1), lambda i: (0, 0)),
            pl.BlockSpec((_H1, _H2), lambda i: (0, 0)),
            pl.BlockSpec((1, _H2), lambda i: (0, 0)),
        ],
        out_specs=[
            pl.BlockSpec((_BLK, _H2), lambda i: (i, 0)),
            pl.BlockSpec((2, _H2), lambda i: (0, 0)),
        ],
        out_shape=[
            jax.ShapeDtypeStruct((_B, _H2), jnp.float32),
            jax.ShapeDtypeStruct((2, _H2), jnp.float32),
        ],
    )(a1, st1, g1, bt1, w2, b2)


def _pass_c(a2, st2, g2, bt2, w3row, fl, c0):
    def body(a_ref, st_ref, g_ref, bt_ref, w_ref, fl_ref, c0_ref, y_ref):
        m = st_ref[0:1, :] * (1.0 / _B)
        v = st_ref[1:2, :] * (1.0 / _B) - m * m
        rstd = lax.rsqrt(v + 1e-5)
        scale = g_ref[...] * rstd
        shift = bt_ref[...] - m * scale
        h = jnp.maximum(a_ref[...] * scale + shift, 0.0)
        mlp = jnp.sum(h * w_ref[...], axis=1, keepdims=True)
        y = mlp + fl_ref[...] + c0_ref[0, 0]
        y_ref[...] = 1.0 / (1.0 + jnp.exp(-y))

    return pl.pallas_call(
        body,
        grid=(_NBLK,),
        in_specs=[
            pl.BlockSpec((_BLK, _H2), lambda i: (i, 0)),
            pl.BlockSpec((2, _H2), lambda i: (0, 0)),
            pl.BlockSpec((1, _H2), lambda i: (0, 0)),
            pl.BlockSpec((1, _H2), lambda i: (0, 0)),
            pl.BlockSpec((1, _H2), lambda i: (0, 0)),
            pl.BlockSpec((_BLK, 1), lambda i: (i, 0)),
            pl.BlockSpec((1, 1), lambda i: (0, 0), memory_space=pltpu.SMEM),
        ],
        out_specs=pl.BlockSpec((_BLK, 1), lambda i: (i, 0)),
        out_shape=jax.ShapeDtypeStruct((_B, 1), jnp.float32),
    )(a2, st2, g2, bt2, w3row, fl, c0)


def kernel(x, embed_table, linear_table, linear_bias,
           W1, b1, g1, bt1, W2, b2, g2, bt2, W3, b3):
    idx = (x.astype(jnp.int32) + jnp.asarray(_OFFSETS)[None, :])
    idxq = ((idx // _TC) * _TC + (idx % _C8) * 8 + (idx % _TC) // _C8)
    idx3 = idxq.reshape(_NW, _NCH, _CH)
    idx3 = jnp.concatenate([idx3, jnp.asarray(_PAD_IDX)], axis=1)

    table_rm = _transpose_table(embed_table.T).reshape(_TROWS * 8, _D)
    o_flat = _sc_gather(idx3, jnp.asarray(_DST3), table_rm)
    o4 = o_flat.reshape(_NBLK4, _B, 128)
    linval = jnp.take(linear_table, idx.reshape(-1), axis=0).reshape(_B, _F)

    w1cat = jnp.concatenate(
        [jnp.concatenate([W1, jnp.asarray(_M)], axis=1),
         jnp.zeros((1, _H1 + _D), jnp.float32)], axis=0)
    w14 = jnp.take(w1cat, jnp.asarray(_KMAP), axis=0)
    a1, fl, st1 = _pass_a(o4, linval, w14, b1.reshape(1, _H1),
                          linear_bias.reshape(1, 1))
    a2, st2 = _pass_b(a1, st1, g1.reshape(1, _H1), bt1.reshape(1, _H1),
                      W2, b2.reshape(1, _H2))
    c0 = b3.reshape(1, 1)
    y = _pass_c(a2, st2, g2.reshape(1, _H2), bt2.reshape(1, _H2),
                W3.reshape(1, _H2), fl, c0)
    return y.reshape(_B)


# speedup vs baseline: 1.1464x; 1.0121x over previous
import functools

import numpy as np
import jax
import jax.numpy as jnp
from jax import lax
from jax.experimental import pallas as pl
from jax.experimental.pallas import tpu as pltpu
from jax.experimental.pallas import tpu_sc as plsc

_FIELD_DIMS = [100000] * 26
_F = 26
_D = 16
_B = 16384
_TOTAL = int(sum(_FIELD_DIMS))
_OFFSETS = np.concatenate(([0], np.cumsum(_FIELD_DIMS)[:-1])).astype(np.int32)

_M = np.zeros((_F * _D, _D), np.float32)
for _f in range(_F):
    for _d in range(_D):
        _M[_f * _D + _d, _d] = 1.0

_NW = 32
_CH = 128
_PER_W = (_B * _F) // _NW
_NCH = _PER_W // _CH

_PAD_IDX = ((np.arange(_NW)[:, None, None] * 409
             + np.arange(2)[None, :, None] * 211
             + np.arange(_CH)[None, None, :] * 97) % 100000).astype(np.int32)

_NBLK4 = 4
_PAD_DST = ((np.arange(_NW)[:, None, None] * 256
             + np.arange(2)[None, :, None] * 128
             + np.arange(_CH)[None, None, :]) * 8 + 7).astype(np.int32)

_LMASK = np.zeros((_NBLK4, 1, 128), np.float32)
for _r in range(_NBLK4):
    for _l in range(8):
        if 4 * _l + _r < _F:
            _LMASK[_r, 0, _l * _D:(_l + 1) * _D] = 1.0

_KMAP = np.full((_NBLK4, 128), _F * _D, np.int32)
for _r in range(_NBLK4):
    for _l in range(8):
        _f2 = 4 * _l + _r
        if _f2 < _F:
            for _d in range(_D):
                _KMAP[_r, _l * _D + _d] = _f2 * _D + _d

_NN = np.arange(_B * _F, dtype=np.int64)
_DSTG = ((_NN % _F % 4) * (_B * 8) + (_NN // _F) * 8
         + (_NN % _F) // 4).astype(np.int32)
_DST3 = np.concatenate([_DSTG.reshape(_NW, _NCH, _CH), _PAD_DST], axis=1)


_TC = 131072
_TG = (_TOTAL + _TC - 1) // _TC
_C8 = _TC // 8
_TROWS = _TG * _C8


def _transpose_table(tab_t):
    def body(x_ref, i_ref, y_ref):
        x = x_ref[...]
        xcat = jnp.concatenate([x[:, u * _C8:(u + 1) * _C8] for u in range(8)],
                               axis=0)
        base = pl.program_id(0) * _TC
        u8 = lax.broadcasted_iota(jnp.int32, (128, _C8), 0) // _D
        col = lax.broadcasted_iota(jnp.int32, (128, _C8), 1)
        xcat = jnp.where(base + u8 * _C8 + col < _TOTAL, xcat, 0.0)
        y_ref[...] = lax.dot_general(xcat, i_ref[...],
                                     (((0,), (0,)), ((), ())),
                                     preferred_element_type=jnp.float32)

    return pl.pallas_call(
        body,
        grid=(_TG,),
        in_specs=[
            pl.BlockSpec((_D, _TC), lambda j: (0, j)),
            pl.BlockSpec((128, 128), lambda j: (0, 0)),
        ],
        out_specs=pl.BlockSpec((_C8, 128), lambda j: (j, 0)),
        out_shape=jax.ShapeDtypeStruct((_TROWS, 128), jnp.float32),
    )(tab_t, jnp.eye(128, dtype=jnp.float32))


def _sc_gather(idx3, dst3, embed_table):
    mesh = plsc.VectorSubcoreMesh(core_axis_name="c", subcore_axis_name="s")

    @functools.partial(
        pl.kernel,
        out_type=jax.ShapeDtypeStruct((_NBLK4 * _B * 8, _D), jnp.float32),
        mesh=mesh,
        compiler_params=pltpu.CompilerParams(use_tc_tiling_on_sc=False),
        scratch_types=[
            pltpu.VMEM((_NCH + 2, _CH), jnp.int32),
            pltpu.VMEM((_NCH + 2, _CH), jnp.int32),
            pltpu.VMEM((2, _CH, _D), jnp.float32),
            pltpu.SemaphoreType.DMA,
            pltpu.SemaphoreType.DMA,
            pltpu.SemaphoreType.DMA,
        ],
    )
    def k(idx_hbm, dst_hbm, tab_hbm, oute_hbm, idx_v, dst_v, ebuf,
          se0, se1, so):
        w = lax.axis_index("s") * 2 + lax.axis_index("c")
        pltpu.sync_copy(idx_hbm.at[w], idx_v)
        pltpu.sync_copy(dst_hbm.at[w], dst_v)
        se = (se0, se1)

        def g_start(j, s):
            pltpu.async_copy(tab_hbm.at[idx_v.at[j]], ebuf.at[s], se[s])

        def g_wait(j, s):
            pltpu.make_async_copy(tab_hbm.at[idx_v.at[j]], ebuf.at[s], se[s]).wait()

        g_start(0, 0)
        g_start(1, 1)

        def body(j2, carry):
            for s in (0, 1):
                j = j2 * 2 + s
                g_wait(j, s)
                pltpu.async_copy(ebuf.at[s], oute_hbm.at[dst_v.at[j]], so).wait()
                g_start(j + 2, s)
            return carry

        lax.fori_loop(0, _NCH // 2, body, 0)
        g_wait(_NCH, 0)
        g_wait(_NCH + 1, 1)

    return k(idx3, dst3, embed_table)


_BLK = 4096
_NBLK = _B // _BLK
_H1 = 128
_H2 = 64
_IN = _F * _D
_INC = _IN + _D


def _pass_a(o4, linval, w14, b1, lbias):
    def body(o_ref, lv_ref, w_ref, b_ref, lb_ref, a1_ref, fl_ref, st_ref):
        i = pl.program_id(0)
        lane = lax.broadcasted_iota(jnp.int32, (_BLK, 128), 1)
        acc = jnp.zeros((_BLK, _H1 + _D), jnp.float32)
        e2 = jnp.zeros((_BLK, 1), jnp.float32)
        for r in range(_NBLK4):
            o = o_ref[r, :, :]
            o = jnp.where(4 * (lane // _D) + r < _F, o, 0.0)
            acc += jnp.dot(o, w_ref[r, :, :],
                           preferred_element_type=jnp.float32)
            e2 += jnp.sum(o * o, axis=1, keepdims=True)
        a1 = acc[:, :_H1] + b_ref[...]
        a1_ref[...] = a1
        s = acc[:, _H1:]
        fm = 0.5 * (jnp.sum(s * s, axis=1, keepdims=True) - e2)
        lin = jnp.sum(lv_ref[...], axis=1, keepdims=True) + lb_ref[0, 0]
        fl_ref[...] = fm + lin

        @pl.when(i == 0)
        def _():
            st_ref[...] = jnp.zeros_like(st_ref)

        st_ref[0:1, :] += jnp.sum(a1, axis=0, keepdims=True)
        st_ref[1:2, :] += jnp.sum(a1 * a1, axis=0, keepdims=True)

    return pl.pallas_call(
        body,
        grid=(_NBLK,),
        in_specs=[
            pl.BlockSpec((_NBLK4, _BLK, 128), lambda i: (0, i, 0)),
            pl.BlockSpec((_BLK, _F), lambda i: (i, 0)),
            pl.BlockSpec((_NBLK4, 128, _H1 + _D), lambda i: (0, 0, 0)),
            pl.BlockSpec((1, _H1), lambda i: (0, 0)),
            pl.BlockSpec((1, 1), lambda i: (0, 0), memory_space=pltpu.SMEM),
        ],
        out_specs=[
            pl.BlockSpec((_BLK, _H1), lambda i: (i, 0)),
            pl.BlockSpec((_BLK, 1), lambda i: (i, 0)),
            pl.BlockSpec((2, _H1), lambda i: (0, 0)),
        ],
        out_shape=[
            jax.ShapeDtypeStruct((_B, _H1), jnp.float32),
            jax.ShapeDtypeStruct((_B, 1), jnp.float32),
            jax.ShapeDtypeStruct((2, _H1), jnp.float32),
        ],
    )(o4, linval, w14, b1, lbias)


def _pass_b(a1, st1, g1, bt1, w2, b2):
    def body(a_ref, st_ref, g_ref, bt_ref, w_ref, b_ref, a2_ref, st2_ref):
        i = pl.program_id(0)
        m = st_ref[0:1, :] * (1.0 / _B)
        v = st_ref[1:2, :] * (1.0 / _B) - m * m
        rstd = lax.rsqrt(v + 1e-5)
        scale = g_ref[...] * rstd
        shift = bt_ref[...] - m * scale
        h = jnp.maximum(a_ref[...] * scale + shift, 0.0)
        a2 = jnp.dot(h, w_ref[...], preferred_element_type=jnp.float32) + b_ref[...]
        a2_ref[...] = a2

        @pl.when(i == 0)
        def _():
            st2_ref[...] = jnp.zeros_like(st2_ref)

        st2_ref[0:1, :] += jnp.sum(a2, axis=0, keepdims=True)
        st2_ref[1:2, :] += jnp.sum(a2 * a2, axis=0, keepdims=True)

    return pl.pallas_call(
        body,
        grid=(_NBLK,),
        in_specs=[
            pl.BlockSpec((_BLK, _H1), lambda i: (i, 0)),
            pl.BlockSpec((2, _H1), lambda i: (0, 0)),
            pl.BlockSpec((1, _H1), lambda i: (0, 0)),
            pl.BlockSpec((1, _H1), lambda i: (0, 0)),
            pl.BlockSpec((_H1, _H2), lambda i: (0, 0)),
            pl.BlockSpec((1, _H2), lambda i: (0, 0)),
        ],
        out_specs=[
            pl.BlockSpec((_BLK, _H2), lambda i: (i, 0)),
            pl.BlockSpec((2, _H2), lambda i: (0, 0)),
        ],
        out_shape=[
            jax.ShapeDtypeStruct((_B, _H2), jnp.float32),
            jax.ShapeDtypeStruct((2, _H2), jnp.float32),
        ],
    )(a1, st1, g1, bt1, w2, b2)


def _pass_c(a2, st2, g2, bt2, w3row, fl, c0):
    def body(a_ref, st_ref, g_ref, bt_ref, w_ref, fl_ref, c0_ref, y_ref):
        m = st_ref[0:1, :] * (1.0 / _B)
        v = st_ref[1:2, :] * (1.0 / _B) - m * m
        rstd = lax.rsqrt(v + 1e-5)
        scale = g_ref[...] * rstd
        shift = bt_ref[...] - m * scale
        h = jnp.maximum(a_ref[...] * scale + shift, 0.0)
        mlp = jnp.sum(h * w_ref[...], axis=1, keepdims=True)
        y = mlp + fl_ref[...] + c0_ref[0, 0]
        y_ref[...] = 1.0 / (1.0 + jnp.exp(-y))

    return pl.pallas_call(
        body,
        grid=(_NBLK,),
        in_specs=[
            pl.BlockSpec((_BLK, _H2), lambda i: (i, 0)),
            pl.BlockSpec((2, _H2), lambda i: (0, 0)),
            pl.BlockSpec((1, _H2), lambda i: (0, 0)),
            pl.BlockSpec((1, _H2), lambda i: (0, 0)),
            pl.BlockSpec((1, _H2), lambda i: (0, 0)),
            pl.BlockSpec((_BLK, 1), lambda i: (i, 0)),
            pl.BlockSpec((1, 1), lambda i: (0, 0), memory_space=pltpu.SMEM),
        ],
        out_specs=pl.BlockSpec((_BLK, 1), lambda i: (i, 0)),
        out_shape=jax.ShapeDtypeStruct((_B, 1), jnp.float32),
    )(a2, st2, g2, bt2, w3row, fl, c0)


def kernel(x, embed_table, linear_table, linear_bias,
           W1, b1, g1, bt1, W2, b2, g2, bt2, W3, b3):
    idx = (x.astype(jnp.int32) + jnp.asarray(_OFFSETS)[None, :])
    idxq = ((idx // _TC) * _TC + (idx % _C8) * 8 + (idx % _TC) // _C8)
    idx3 = idxq.reshape(_NW, _NCH, _CH)
    idx3 = jnp.concatenate([idx3, jnp.asarray(_PAD_IDX)], axis=1)

    table_rm = _transpose_table(embed_table.T).reshape(_TROWS * 8, _D)
    o_flat = _sc_gather(idx3, jnp.asarray(_DST3), table_rm)
    o4 = o_flat.reshape(_NBLK4, _B, 128)
    linval = jnp.take(linear_table, idx.reshape(-1), axis=0).reshape(_B, _F)

    w1cat = jnp.concatenate(
        [jnp.concatenate([W1, jnp.asarray(_M)], axis=1),
         jnp.zeros((1, _H1 + _D), jnp.float32)], axis=0)
    w14 = jnp.take(w1cat, jnp.asarray(_KMAP), axis=0)
    a1, fl, st1 = _pass_a(o4, linval, w14, b1.reshape(1, _H1),
                          linear_bias.reshape(1, 1))
    a2, st2 = _pass_b(a1, st1, g1.reshape(1, _H1), bt1.reshape(1, _H1),
                      W2, b2.reshape(1, _H2))
    c0 = b3.reshape(1, 1)
    y = _pass_c(a2, st2, g2.reshape(1, _H2), bt2.reshape(1, _H2),
                W3.reshape(1, _H2), fl, c0)
    return y.reshape(_B)


# speedup vs baseline: 1.5234x; 1.3289x over previous
import functools

import numpy as np
import jax
import jax.numpy as jnp
from jax import lax
from jax.experimental import pallas as pl
from jax.experimental.pallas import tpu as pltpu
from jax.experimental.pallas import tpu_sc as plsc

_FIELD_DIMS = [100000] * 26
_F = 26
_D = 16
_B = 16384
_TOTAL = int(sum(_FIELD_DIMS))
_OFFSETS = np.concatenate(([0], np.cumsum(_FIELD_DIMS)[:-1])).astype(np.int32)

_M = np.zeros((_F * _D, _D), np.float32)
for _f in range(_F):
    for _d in range(_D):
        _M[_f * _D + _d, _d] = 1.0

_NW = 32
_CH = 128
_PER_W = (_B * _F) // _NW
_NCH = _PER_W // _CH

_PAD_IDX = ((np.arange(_NW)[:, None, None] * 409
             + np.arange(2)[None, :, None] * 211
             + np.arange(_CH)[None, None, :] * 97) % 100000).astype(np.int32)

_NBLK4 = 4
_PAD_DST = ((np.arange(_NW)[:, None, None] * 256
             + np.arange(2)[None, :, None] * 128
             + np.arange(_CH)[None, None, :]) * 8 + 7).astype(np.int32)

_LMASK = np.zeros((_NBLK4, 1, 128), np.float32)
for _r in range(_NBLK4):
    for _l in range(8):
        if 4 * _l + _r < _F:
            _LMASK[_r, 0, _l * _D:(_l + 1) * _D] = 1.0

_KMAP = np.full((_NBLK4, 128), _F * _D, np.int32)
for _r in range(_NBLK4):
    for _l in range(8):
        _f2 = 4 * _l + _r
        if _f2 < _F:
            for _d in range(_D):
                _KMAP[_r, _l * _D + _d] = _f2 * _D + _d

_NN = np.arange(_B * _F, dtype=np.int64)
_DSTG = ((_NN % _F % 4) * (_B * 8) + (_NN // _F) * 8
         + (_NN % _F) // 4).astype(np.int32)
_DST3 = np.concatenate([_DSTG.reshape(_NW, _NCH, _CH), _PAD_DST], axis=1)


_TC = 131072
_TG = (_TOTAL + _TC - 1) // _TC
_C8 = _TC // 8
_TROWS = _TG * _C8


def _transpose_table(tab_t):
    def body(x_ref, i_ref, y_ref):
        x = x_ref[...]
        xcat = jnp.concatenate([x[:, u * _C8:(u + 1) * _C8] for u in range(8)],
                               axis=0)
        base = pl.program_id(0) * _TC
        u8 = lax.broadcasted_iota(jnp.int32, (128, _C8), 0) // _D
        col = lax.broadcasted_iota(jnp.int32, (128, _C8), 1)
        xcat = jnp.where(base + u8 * _C8 + col < _TOTAL, xcat, 0.0)
        y_ref[...] = lax.dot_general(xcat, i_ref[...],
                                     (((0,), (0,)), ((), ())),
                                     preferred_element_type=jnp.float32)

    return pl.pallas_call(
        body,
        grid=(_TG,),
        in_specs=[
            pl.BlockSpec((_D, _TC), lambda j: (0, j)),
            pl.BlockSpec((128, 128), lambda j: (0, 0)),
        ],
        out_specs=pl.BlockSpec((_C8, 128), lambda j: (j, 0)),
        out_shape=jax.ShapeDtypeStruct((_TROWS, 128), jnp.float32),
    )(tab_t, jnp.eye(128, dtype=jnp.float32))


_TL = 65536
_TLG = (_TOTAL + _TL - 1) // _TL


def _squeeze_lin(lin_t):
    def body(x_ref, y_ref):
        y_ref[...] = x_ref[0, :]

    return pl.pallas_call(
        body,
        grid=(_TLG,),
        in_specs=[pl.BlockSpec((1, _TL), lambda j: (0, j))],
        out_specs=pl.BlockSpec((_TL,), lambda j: (j,)),
        out_shape=jax.ShapeDtypeStruct((_TOTAL,), jnp.float32),
    )(lin_t)


def _sc_gather(idx3, dst3, embed_table):
    mesh = plsc.VectorSubcoreMesh(core_axis_name="c", subcore_axis_name="s")

    @functools.partial(
        pl.kernel,
        out_type=jax.ShapeDtypeStruct((_NBLK4 * _B * 8, _D), jnp.float32),
        mesh=mesh,
        compiler_params=pltpu.CompilerParams(use_tc_tiling_on_sc=False),
        scratch_types=[
            pltpu.VMEM((_NCH + 2, _CH), jnp.int32),
            pltpu.VMEM((_NCH + 2, _CH), jnp.int32),
            pltpu.VMEM((2, _CH, _D), jnp.float32),
            pltpu.SemaphoreType.DMA,
            pltpu.SemaphoreType.DMA,
            pltpu.SemaphoreType.DMA,
        ],
    )
    def k(idx_hbm, dst_hbm, tab_hbm, oute_hbm, idx_v, dst_v, ebuf,
          se0, se1, so):
        w = lax.axis_index("s") * 2 + lax.axis_index("c")
        pltpu.sync_copy(idx_hbm.at[w], idx_v)
        pltpu.sync_copy(dst_hbm.at[w], dst_v)
        se = (se0, se1)

        def g_start(j, s):
            pltpu.async_copy(tab_hbm.at[idx_v.at[j]], ebuf.at[s], se[s])

        def g_wait(j, s):
            pltpu.make_async_copy(tab_hbm.at[idx_v.at[j]], ebuf.at[s], se[s]).wait()

        g_start(0, 0)
        g_start(1, 1)

        def body(j2, carry):
            for s in (0, 1):
                j = j2 * 2 + s
                g_wait(j, s)
                pltpu.async_copy(ebuf.at[s], oute_hbm.at[dst_v.at[j]], so).wait()
                g_start(j + 2, s)
            return carry

        lax.fori_loop(0, _NCH // 2, body, 0)
        g_wait(_NCH, 0)
        g_wait(_NCH + 1, 1)

    return k(idx3, dst3, embed_table)


_BLK = 4096
_NBLK = _B // _BLK
_H1 = 128
_H2 = 64
_IN = _F * _D
_INC = _IN + _D


def _pass_a(o4, linval, w14, b1, lbias):
    def body(o_ref, lv_ref, w_ref, b_ref, lb_ref, a1_ref, fl_ref, st_ref):
        i = pl.program_id(0)
        lane = lax.broadcasted_iota(jnp.int32, (_BLK, 128), 1)
        acc = jnp.zeros((_BLK, _H1 + _D), jnp.float32)
        e2 = jnp.zeros((_BLK, 1), jnp.float32)
        for r in range(_NBLK4):
            o = o_ref[r, :, :]
            o = jnp.where(4 * (lane // _D) + r < _F, o, 0.0)
            acc += jnp.dot(o, w_ref[r, :, :],
                           preferred_element_type=jnp.float32)
            e2 += jnp.sum(o * o, axis=1, keepdims=True)
        a1 = acc[:, :_H1] + b_ref[...]
        a1_ref[...] = a1
        s = acc[:, _H1:]
        fm = 0.5 * (jnp.sum(s * s, axis=1, keepdims=True) - e2)
        lin = jnp.sum(lv_ref[...], axis=1, keepdims=True) + lb_ref[0, 0]
        fl_ref[...] = fm + lin

        @pl.when(i == 0)
        def _():
            st_ref[...] = jnp.zeros_like(st_ref)

        st_ref[0:1, :] += jnp.sum(a1, axis=0, keepdims=True)
        st_ref[1:2, :] += jnp.sum(a1 * a1, axis=0, keepdims=True)

    return pl.pallas_call(
        body,
        grid=(_NBLK,),
        in_specs=[
            pl.BlockSpec((_NBLK4, _BLK, 128), lambda i: (0, i, 0)),
            pl.BlockSpec((_BLK, _F), lambda i: (i, 0)),
            pl.BlockSpec((_NBLK4, 128, _H1 + _D), lambda i: (0, 0, 0)),
            pl.BlockSpec((1, _H1), lambda i: (0, 0)),
            pl.BlockSpec((1, 1), lambda i: (0, 0), memory_space=pltpu.SMEM),
        ],
        out_specs=[
            pl.BlockSpec((_BLK, _H1), lambda i: (i, 0)),
            pl.BlockSpec((_BLK, 1), lambda i: (i, 0)),
            pl.BlockSpec((2, _H1), lambda i: (0, 0)),
        ],
        out_shape=[
            jax.ShapeDtypeStruct((_B, _H1), jnp.float32),
            jax.ShapeDtypeStruct((_B, 1), jnp.float32),
            jax.ShapeDtypeStruct((2, _H1), jnp.float32),
        ],
    )(o4, linval, w14, b1, lbias)


def _pass_b(a1, st1, g1, bt1, w2, b2):
    def body(a_ref, st_ref, g_ref, bt_ref, w_ref, b_ref, a2_ref, st2_ref):
        i = pl.program_id(0)
        m = st_ref[0:1, :] * (1.0 / _B)
        v = st_ref[1:2, :] * (1.0 / _B) - m * m
        rstd = lax.rsqrt(v + 1e-5)
        scale = g_ref[...] * rstd
        shift = bt_ref[...] - m * scale
        h = jnp.maximum(a_ref[...] * scale + shift, 0.0)
        a2 = jnp.dot(h, w_ref[...], preferred_element_type=jnp.float32) + b_ref[...]
        a2_ref[...] = a2

        @pl.when(i == 0)
        def _():
            st2_ref[...] = jnp.zeros_like(st2_ref)

        st2_ref[0:1, :] += jnp.sum(a2, axis=0, keepdims=True)
        st2_ref[1:2, :] += jnp.sum(a2 * a2, axis=0, keepdims=True)

    return pl.pallas_call(
        body,
        grid=(_NBLK,),
        in_specs=[
            pl.BlockSpec((_BLK, _H1), lambda i: (i, 0)),
            pl.BlockSpec((2, _H1), lambda i: (0, 0)),
            pl.BlockSpec((1, _H1), lambda i: (0, 0)),
            pl.BlockSpec((1, _H1), lambda i: (0, 0)),
            pl.BlockSpec((_H1, _H2), lambda i: (0, 0)),
            pl.BlockSpec((1, _H2), lambda i: (0, 0)),
        ],
        out_specs=[
            pl.BlockSpec((_BLK, _H2), lambda i: (i, 0)),
            pl.BlockSpec((2, _H2), lambda i: (0, 0)),
        ],
        out_shape=[
            jax.ShapeDtypeStruct((_B, _H2), jnp.float32),
            jax.ShapeDtypeStruct((2, _H2), jnp.float32),
        ],
    )(a1, st1, g1, bt1, w2, b2)


def _pass_c(a2, st2, g2, bt2, w3row, fl, c0):
    def body(a_ref, st_ref, g_ref, bt_ref, w_ref, fl_ref, c0_ref, y_ref):
        m = st_ref[0:1, :] * (1.0 / _B)
        v = st_ref[1:2, :] * (1.0 / _B) - m * m
        rstd = lax.rsqrt(v + 1e-5)
        scale = g_ref[...] * rstd
        shift = bt_ref[...] - m * scale
        h = jnp.maximum(a_ref[...] * scale + shift, 0.0)
        mlp = jnp.sum(h * w_ref[...], axis=1, keepdims=True)
        y = mlp + fl_ref[...] + c0_ref[0, 0]
        y_ref[...] = 1.0 / (1.0 + jnp.exp(-y))

    return pl.pallas_call(
        body,
        grid=(_NBLK,),
        in_specs=[
            pl.BlockSpec((_BLK, _H2), lambda i: (i, 0)),
            pl.BlockSpec((2, _H2), lambda i: (0, 0)),
            pl.BlockSpec((1, _H2), lambda i: (0, 0)),
            pl.BlockSpec((1, _H2), lambda i: (0, 0)),
            pl.BlockSpec((1, _H2), lambda i: (0, 0)),
            pl.BlockSpec((_BLK, 1), lambda i: (i, 0)),
            pl.BlockSpec((1, 1), lambda i: (0, 0), memory_space=pltpu.SMEM),
        ],
        out_specs=pl.BlockSpec((_BLK, 1), lambda i: (i, 0)),
        out_shape=jax.ShapeDtypeStruct((_B, 1), jnp.float32),
    )(a2, st2, g2, bt2, w3row, fl, c0)


def kernel(x, embed_table, linear_table, linear_bias,
           W1, b1, g1, bt1, W2, b2, g2, bt2, W3, b3):
    idx = (x.astype(jnp.int32) + jnp.asarray(_OFFSETS)[None, :])
    idxq = ((idx // _TC) * _TC + (idx % _C8) * 8 + (idx % _TC) // _C8)
    idx3 = idxq.reshape(_NW, _NCH, _CH)
    idx3 = jnp.concatenate([idx3, jnp.asarray(_PAD_IDX)], axis=1)

    table_rm = _transpose_table(embed_table.T).reshape(_TROWS * 8, _D)
    o_flat = _sc_gather(idx3, jnp.asarray(_DST3), table_rm)
    o4 = o_flat.reshape(_NBLK4, _B, 128)
    lin1d = _squeeze_lin(linear_table.T)
    linval = jnp.take(lin1d, idx.reshape(-1), axis=0,
                      mode="clip").reshape(_B, _F)

    w1cat = jnp.concatenate(
        [jnp.concatenate([W1, jnp.asarray(_M)], axis=1),
         jnp.zeros((1, _H1 + _D), jnp.float32)], axis=0)
    w14 = jnp.take(w1cat, jnp.asarray(_KMAP), axis=0)
    a1, fl, st1 = _pass_a(o4, linval, w14, b1.reshape(1, _H1),
                          linear_bias.reshape(1, 1))
    a2, st2 = _pass_b(a1, st1, g1.reshape(1, _H1), bt1.reshape(1, _H1),
                      W2, b2.reshape(1, _H2))
    c0 = b3.reshape(1, 1)
    y = _pass_c(a2, st2, g2.reshape(1, _H2), bt2.reshape(1, _H2),
                W3.reshape(1, _H2), fl, c0)
    return y.reshape(_B)
